# baseline (speedup 1.0000x reference)
import jax
import jax.numpy as jnp
from jax.experimental import pallas as pl
from jax.experimental.pallas import tpu as pltpu

NUM_TOKENS = 32768
HIDDEN = 4096
NUM_EXPERTS = 64
TOP_K = 8
BLOCK_T = 1024


def _router_block(x_ref, w_ref, weights_ref, idx_ref):
    x = x_ref[...]
    w = w_ref[...]
    logits = jax.lax.dot_general(
        x, w, (((1,), (1,)), ((), ())), preferred_element_type=jnp.float32
    )
    cols = jax.lax.broadcasted_iota(jnp.int32, logits.shape, 1).astype(jnp.float32)
    m = jnp.max(logits, axis=-1, keepdims=True)
    work = logits
    top_l = []
    top_i = []
    for k in range(TOP_K):
        cur = m if k == 0 else jnp.max(work, axis=-1, keepdims=True)
        hit = work == cur
        idx = jnp.min(
            jnp.where(hit, cols, float(NUM_EXPERTS)), axis=-1, keepdims=True
        )
        top_l.append(cur)
        top_i.append(idx)
        work = jnp.where(cols == idx, -jnp.inf, work)

    unnorm = jnp.exp(jnp.concatenate(top_l, axis=-1) - m)
    weights_ref[...] = unnorm / jnp.sum(unnorm, axis=-1, keepdims=True)
    idx_ref[...] = jnp.concatenate(top_i, axis=-1).astype(jnp.int32)


def kernel(hidden_states, gate_weight):
    grid = (NUM_TOKENS // BLOCK_T,)
    out_shapes = (
        jax.ShapeDtypeStruct((NUM_TOKENS, TOP_K), jnp.float32),
        jax.ShapeDtypeStruct((NUM_TOKENS, TOP_K), jnp.int32),
    )
    return pl.pallas_call(
        _router_block,
        grid=grid,
        in_specs=[
            pl.BlockSpec((BLOCK_T, HIDDEN), lambda i: (i, 0)),
            pl.BlockSpec((NUM_EXPERTS, HIDDEN), lambda i: (0, 0)),
        ],
        out_specs=(
            pl.BlockSpec((BLOCK_T, TOP_K), lambda i: (i, 0)),
            pl.BlockSpec((BLOCK_T, TOP_K), lambda i: (i, 0)),
        ),
        out_shape=out_shapes,
        compiler_params=pltpu.CompilerParams(
            dimension_semantics=("arbitrary",),
        ),
    )(hidden_states, gate_weight)
